# baseline (speedup 1.0000x reference)
import jax
import jax.numpy as jnp
from jax.experimental import pallas as pl

_MEMORY_COEF = 0.25
_DHAT_COEF = 0.1


def _make_kernel_body(rec_scale, mem_scale):
    def body(xhat_ref, x_ref, h_ref, m_ref, dhat_ref, out_ref):
        i = pl.program_id(0)

        diff = xhat_ref[0] - x_ref[0]
        rec_part = jnp.sum(diff * diff)

        h = h_ref[0]
        m = m_ref[...]
        cross = jax.lax.dot_general(
            h, m, (((0,), (0,)), ((), ())), preferred_element_type=jnp.float32
        )
        m_sq = jnp.sum(m * m, axis=0)
        h_sq = jnp.sum(h * h, axis=0)
        dmin = jnp.min(m_sq[None, :] - 2.0 * cross, axis=1)
        mem_part = jnp.sum(jnp.maximum(h_sq + dmin, 0.0))

        part = rec_part * rec_scale + mem_part * mem_scale

        @pl.when(i == 0)
        def _init():
            out_ref[...] = jnp.full((1, 1), -_DHAT_COEF, jnp.float32) * jnp.mean(
                dhat_ref[...], keepdims=True
            )

        out_ref[...] += jnp.reshape(part, (1, 1))

    return body


def kernel(Xhat, X, H, M, Dhat):
    B, d, T = H.shape
    K = M.shape[1]
    n_rec = Xhat.size
    chunk = n_rec // B

    rows = chunk // 1024
    Xhat2 = Xhat.reshape(B, rows, 1024)
    X2 = X.reshape(B, rows, 1024)
    Dhat2 = Dhat.reshape(1, B)

    rec_scale = 1.0 / float(n_rec)
    mem_scale = _MEMORY_COEF * 2.0 / float(B * d * T)

    out = pl.pallas_call(
        _make_kernel_body(rec_scale, mem_scale),
        grid=(B,),
        in_specs=[
            pl.BlockSpec((1, rows, 1024), lambda i: (i, 0, 0)),
            pl.BlockSpec((1, rows, 1024), lambda i: (i, 0, 0)),
            pl.BlockSpec((1, d, T), lambda i: (i, 0, 0)),
            pl.BlockSpec((d, K), lambda i: (0, 0)),
            pl.BlockSpec((1, B), lambda i: (0, 0)),
        ],
        out_specs=pl.BlockSpec((1, 1), lambda i: (0, 0)),
        out_shape=jax.ShapeDtypeStruct((1, 1), jnp.float32),
    )(Xhat2, X2, H, M, Dhat2)
    return out[0, 0]
